# baseline (speedup 1.0000x reference)
import functools

import jax
import jax.numpy as jnp
from jax import lax
from jax.experimental import pallas as pl
from jax.experimental.pallas import tpu as pltpu

ITEM_TILE = 1024


def _matmul_body(u_ref, it_ref, out_ref, *, n_items):
    t = pl.program_id(0)
    s = lax.dot_general(
        u_ref[...], it_ref[...],
        (((1,), (1,)), ((), ())),
        preferred_element_type=jnp.float32,
    )
    col = t * ITEM_TILE + lax.broadcasted_iota(jnp.int32, s.shape, 1)
    out_ref[...] = jnp.where(col < n_items, s, jnp.float32(-1e30))


def _scores_pallas(u, items_pad, n_items):
    q, d = u.shape
    n_pad = items_pad.shape[0]
    grid = (n_pad // ITEM_TILE,)
    return pl.pallas_call(
        functools.partial(_matmul_body, n_items=n_items),
        grid=grid,
        in_specs=[
            pl.BlockSpec((q, d), lambda t: (0, 0)),
            pl.BlockSpec((ITEM_TILE, d), lambda t: (t, 0)),
        ],
        out_specs=pl.BlockSpec((q, ITEM_TILE), lambda t: (0, t)),
        out_shape=jax.ShapeDtypeStruct((q, n_pad), jnp.float32),
    )(u, items_pad)


def kernel(uids, topk, user_embs, item_embs):
    n_items, d = item_embs.shape
    n_pad = ((n_items + ITEM_TILE - 1) // ITEM_TILE) * ITEM_TILE
    items_pad = jnp.pad(item_embs, ((0, n_pad - n_items), (0, 0)))
    u = jnp.take(user_embs, uids, axis=0)
    scores = _scores_pallas(u, items_pad, n_items)
    top_vals, top_idx = lax.top_k(scores, 100)
    top_idx = top_idx + jnp.asarray(topk - topk, dtype=top_idx.dtype)
    return top_vals, top_idx


# speedup vs baseline: 5.8391x; 5.8391x over previous
import functools

import jax
import jax.numpy as jnp
from jax import lax
from jax.experimental import pallas as pl
from jax.experimental.pallas import tpu as pltpu

ITEM_TILE = 2048
ROW_TILE = 256
LEAF = 16
K = 100


def _matmul_body(u_ref, it_ref, out_ref, bmax_ref, *, n_items):
    t = pl.program_id(1)
    s = lax.dot_general(
        u_ref[...], it_ref[...],
        (((1,), (1,)), ((), ())),
        preferred_element_type=jnp.float32,
    )
    col = t * ITEM_TILE + lax.broadcasted_iota(jnp.int32, s.shape, 1)
    s = jnp.where(col < n_items, s, jnp.float32(-1e30))
    out_ref[...] = s
    q = s.shape[0]
    bmax_ref[...] = jnp.max(s.reshape(q, ITEM_TILE // LEAF, LEAF), axis=-1)


def _scores_pallas(u, items_pad, n_items):
    q, d = u.shape
    n_pad = items_pad.shape[0]
    grid = (q // ROW_TILE, n_pad // ITEM_TILE)
    return pl.pallas_call(
        functools.partial(_matmul_body, n_items=n_items),
        grid=grid,
        in_specs=[
            pl.BlockSpec((ROW_TILE, d), lambda r, t: (r, 0)),
            pl.BlockSpec((ITEM_TILE, d), lambda r, t: (t, 0)),
        ],
        out_specs=[
            pl.BlockSpec((ROW_TILE, ITEM_TILE), lambda r, t: (r, t)),
            pl.BlockSpec((ROW_TILE, ITEM_TILE // LEAF), lambda r, t: (r, t)),
        ],
        out_shape=[
            jax.ShapeDtypeStruct((q, n_pad), jnp.float32),
            jax.ShapeDtypeStruct((q, n_pad // LEAF), jnp.float32),
        ],
    )(u, items_pad)


def kernel(uids, topk, user_embs, item_embs):
    n_items, d = item_embs.shape
    q = uids.shape[0]
    n_pad = ((n_items + ITEM_TILE - 1) // ITEM_TILE) * ITEM_TILE
    items_pad = jnp.pad(item_embs, ((0, n_pad - n_items), (0, 0)))
    u = jnp.take(user_embs, uids, axis=0)
    scores, bmax = _scores_pallas(u, items_pad, n_items)

    _, bids = lax.top_k(bmax, K)
    cand_idx = (bids[:, :, None] * LEAF
                + jnp.arange(LEAF, dtype=bids.dtype)[None, None, :]
                ).reshape(q, K * LEAF)
    cand = jnp.take_along_axis(scores, cand_idx, axis=1)
    top_vals, pos = lax.top_k(cand, K)
    top_idx = jnp.take_along_axis(cand_idx, pos, axis=1)
    _, top_idx, top_vals = lax.sort(
        (-top_vals, top_idx, top_vals), dimension=1, num_keys=2)
    top_idx = top_idx + jnp.asarray(topk - topk, dtype=top_idx.dtype)
    return top_vals, top_idx


# speedup vs baseline: 16.5095x; 2.8274x over previous
import functools

import jax
import jax.numpy as jnp
from jax import lax
from jax.experimental import pallas as pl
from jax.experimental.pallas import tpu as pltpu

ITEM_TILE = 2048
ROW_TILE = 256
K = 100
NEG = -3.0e38
SPAD = -1.0e30
NT_PAD = 64


def _matmul_body(u_ref, it_ref, out_ref, bmax_ref, *, n_items, n_tiles):
    t = pl.program_id(1)

    @pl.when(t < n_tiles)
    def _():
        s = lax.dot_general(
            u_ref[...], it_ref[...],
            (((1,), (1,)), ((), ())),
            preferred_element_type=jnp.float32,
        )
        col = t * ITEM_TILE + lax.broadcasted_iota(jnp.int32, s.shape, 1)
        s = jnp.where(col < n_items, s, SPAD)
        out_ref[...] = s
        b = s
        for half in (1024, 512, 256, 128):
            b = jnp.maximum(b[:, :half], b[:, half:])
        bmax_ref[...] = b

    @pl.when(t >= n_tiles)
    def _():
        bmax_ref[...] = jnp.full(bmax_ref.shape, NEG, jnp.float32)


def _scores_pallas(u, items_pad, n_items):
    q, d = u.shape
    n_pad = items_pad.shape[0]
    n_tiles = n_pad // ITEM_TILE
    grid = (q // ROW_TILE, NT_PAD)
    return pl.pallas_call(
        functools.partial(_matmul_body, n_items=n_items, n_tiles=n_tiles),
        grid=grid,
        in_specs=[
            pl.BlockSpec((ROW_TILE, d), lambda r, t: (r, 0)),
            pl.BlockSpec((ITEM_TILE, d),
                         lambda r, t: (jnp.minimum(t, n_tiles - 1), 0)),
        ],
        out_specs=[
            pl.BlockSpec((ROW_TILE, ITEM_TILE),
                         lambda r, t: (r, jnp.minimum(t, n_tiles - 1))),
            pl.BlockSpec((ROW_TILE, 128), lambda r, t: (r, t)),
        ],
        out_shape=[
            jax.ShapeDtypeStruct((q, n_pad), jnp.float32),
            jax.ShapeDtypeStruct((q, NT_PAD * 128), jnp.float32),
        ],
    )(u, items_pad)


def _peel(x, k, fold_to=None, with_vals=False):
    q, n = x.shape

    def body(x_ref, *out_refs):
        v = x_ref[...]
        m = n
        if fold_to is not None:
            while m > fold_to:
                m //= 2
                v = jnp.maximum(v[:, :m], v[:, m:])
        r = v.shape[0]
        col = lax.broadcasted_iota(jnp.int32, (r, m), 1)
        ocol = lax.broadcasted_iota(jnp.int32, (r, 128), 1)

        def step(i, carry):
            v, acc, accv = carry
            mx = jnp.max(v, axis=1, keepdims=True)
            am = jnp.min(jnp.where(v == mx, col, m), axis=1, keepdims=True)
            acc = jnp.where(ocol == i, am, acc)
            accv = jnp.where(ocol == i, mx, accv)
            v = jnp.where(col == am, NEG, v)
            return v, acc, accv

        _, acc, accv = lax.fori_loop(
            0, k, step,
            (v, jnp.zeros((r, 128), jnp.int32),
             jnp.full((r, 128), NEG, jnp.float32)))
        out_refs[0][...] = acc
        if with_vals:
            out_refs[1][...] = accv

    out_shape = [jax.ShapeDtypeStruct((q, 128), jnp.int32)]
    out_specs = [pl.BlockSpec((ROW_TILE, 128), lambda r: (r, 0))]
    if with_vals:
        out_shape.append(jax.ShapeDtypeStruct((q, 128), jnp.float32))
        out_specs.append(pl.BlockSpec((ROW_TILE, 128), lambda r: (r, 0)))
    res = pl.pallas_call(
        body,
        grid=(q // ROW_TILE,),
        in_specs=[pl.BlockSpec((ROW_TILE, n), lambda r: (r, 0))],
        out_specs=out_specs,
        out_shape=out_shape,
    )(x)
    return res if with_vals else (res[0],)


def _pad_lanes(x, n_to):
    return jnp.pad(x, ((0, 0), (0, n_to - x.shape[1])),
                   constant_values=float(NEG))


def kernel(uids, topk, user_embs, item_embs):
    n_items, d = item_embs.shape
    q = uids.shape[0]
    n_pad = ((n_items + ITEM_TILE - 1) // ITEM_TILE) * ITEM_TILE
    items_pad = jnp.pad(item_embs, ((0, n_pad - n_items), (0, 0)))
    u = jnp.take(user_embs, uids, axis=0)
    scores, bmax = _scores_pallas(u, items_pad, n_items)

    (g_ids,) = _peel(bmax, K, fold_to=512)
    g_ids = g_ids[:, :K]

    j16 = jnp.arange(16, dtype=jnp.int32) * 512
    lids = (g_ids[:, :, None] + j16[None, None, :]).reshape(q, K * 16)
    cand1 = jnp.take_along_axis(bmax, lids, axis=1)
    (p1,) = _peel(_pad_lanes(cand1, 1664), K)
    leaf = jnp.take_along_axis(lids, p1[:, :K], axis=1)

    base = (leaf // 128) * ITEM_TILE + (leaf % 128)
    j128 = jnp.arange(16, dtype=jnp.int32) * 128
    iidx = (base[:, :, None] + j128[None, None, :]).reshape(q, K * 16)
    cand2 = jnp.take_along_axis(scores, iidx, axis=1)
    p2, vals = _peel(_pad_lanes(cand2, 1664), K, with_vals=True)
    top_vals = vals[:, :K]
    top_idx = jnp.take_along_axis(iidx, p2[:, :K], axis=1)

    _, top_idx, top_vals = lax.sort(
        (-top_vals, top_idx, top_vals), dimension=1, num_keys=2)
    top_idx = top_idx + jnp.asarray(topk - topk, dtype=top_idx.dtype)
    return top_vals, top_idx


# speedup vs baseline: 17.3006x; 1.0479x over previous
import functools

import jax
import jax.numpy as jnp
from jax import lax
from jax.experimental import pallas as pl
from jax.experimental.pallas import tpu as pltpu

ITEM_TILE = 2048
ROW_TILE = 256
K = 100
NEG = -3.0e38
SPAD = -1.0e30
NT_PAD = 64


def _matmul_body(u_ref, it_ref, out_ref, bmax_ref, *, n_items, n_tiles):
    t = pl.program_id(1)

    @pl.when(t < n_tiles)
    def _():
        s = lax.dot_general(
            u_ref[...], it_ref[...],
            (((1,), (1,)), ((), ())),
            preferred_element_type=jnp.float32,
        )
        col = t * ITEM_TILE + lax.broadcasted_iota(jnp.int32, s.shape, 1)
        s = jnp.where(col < n_items, s, SPAD)
        out_ref[...] = s
        b = s
        for half in (1024, 512, 256, 128):
            b = jnp.maximum(b[:, :half], b[:, half:])
        bmax_ref[...] = b

    @pl.when(t >= n_tiles)
    def _():
        bmax_ref[...] = jnp.full(bmax_ref.shape, NEG, jnp.float32)


def _scores_pallas(u, items_pad, n_items):
    q, d = u.shape
    n_pad = items_pad.shape[0]
    n_tiles = n_pad // ITEM_TILE
    grid = (q // ROW_TILE, NT_PAD)
    return pl.pallas_call(
        functools.partial(_matmul_body, n_items=n_items, n_tiles=n_tiles),
        grid=grid,
        in_specs=[
            pl.BlockSpec((ROW_TILE, d), lambda r, t: (r, 0)),
            pl.BlockSpec((ITEM_TILE, d),
                         lambda r, t: (jnp.minimum(t, n_tiles - 1), 0)),
        ],
        out_specs=[
            pl.BlockSpec((ROW_TILE, ITEM_TILE),
                         lambda r, t: (r, jnp.minimum(t, n_tiles - 1))),
            pl.BlockSpec((ROW_TILE, 128), lambda r, t: (r, t)),
        ],
        out_shape=[
            jax.ShapeDtypeStruct((q, n_pad), jnp.float32),
            jax.ShapeDtypeStruct((q, NT_PAD * 128), jnp.float32),
        ],
    )(u, items_pad)


def _peel(x, k, fold_to=None, with_vals=False, row_block=None):
    q, n = x.shape

    def body(x_ref, *out_refs):
        v = x_ref[...]
        m = n
        if fold_to is not None:
            while m > fold_to:
                m //= 2
                v = jnp.maximum(v[:, :m], v[:, m:])
        r = v.shape[0]
        col = lax.broadcasted_iota(jnp.int32, (r, m), 1)
        ocol = lax.broadcasted_iota(jnp.int32, (r, 128), 1)

        def step(i, carry):
            v, acc, accv = carry
            mx = jnp.max(v, axis=1, keepdims=True)
            am = jnp.min(jnp.where(v == mx, col, m), axis=1, keepdims=True)
            acc = jnp.where(ocol == i, am, acc)
            accv = jnp.where(ocol == i, mx, accv)
            v = jnp.where(col == am, NEG, v)
            return v, acc, accv

        _, acc, accv = lax.fori_loop(
            0, k, step,
            (v, jnp.zeros((r, 128), jnp.int32),
             jnp.full((r, 128), NEG, jnp.float32)))
        out_refs[0][...] = acc
        if with_vals:
            out_refs[1][...] = accv

    rb = row_block or q
    out_shape = [jax.ShapeDtypeStruct((q, 128), jnp.int32)]
    out_specs = [pl.BlockSpec((rb, 128), lambda r: (r, 0))]
    if with_vals:
        out_shape.append(jax.ShapeDtypeStruct((q, 128), jnp.float32))
        out_specs.append(pl.BlockSpec((rb, 128), lambda r: (r, 0)))
    res = pl.pallas_call(
        body,
        grid=(q // rb,),
        in_specs=[pl.BlockSpec((rb, n), lambda r: (r, 0))],
        out_specs=out_specs,
        out_shape=out_shape,
    )(x)
    return res if with_vals else (res[0],)


def _pad_lanes(x, n_to):
    return jnp.pad(x, ((0, 0), (0, n_to - x.shape[1])),
                   constant_values=float(NEG))


def kernel(uids, topk, user_embs, item_embs):
    n_items, d = item_embs.shape
    q = uids.shape[0]
    n_pad = ((n_items + ITEM_TILE - 1) // ITEM_TILE) * ITEM_TILE
    items_pad = jnp.pad(item_embs, ((0, n_pad - n_items), (0, 0)))
    u = jnp.take(user_embs, uids, axis=0)
    scores, bmax = _scores_pallas(u, items_pad, n_items)

    (g_ids,) = _peel(bmax, K, fold_to=512, row_block=512)
    g_ids = g_ids[:, :K]

    j16 = jnp.arange(16, dtype=jnp.int32) * 512
    lids = (g_ids[:, :, None] + j16[None, None, :]).reshape(q, K * 16)
    cand1 = jnp.take_along_axis(bmax, lids, axis=1)
    (p1,) = _peel(_pad_lanes(cand1, 1664), K)
    leaf = jnp.take_along_axis(lids, p1[:, :K], axis=1)

    base = (leaf // 128) * ITEM_TILE + (leaf % 128)
    j128 = jnp.arange(16, dtype=jnp.int32) * 128
    iidx = (base[:, :, None] + j128[None, None, :]).reshape(q, K * 16)
    cand2 = jnp.take_along_axis(scores, iidx, axis=1)
    p2, vals = _peel(_pad_lanes(cand2, 1664), K, with_vals=True)
    top_vals = vals[:, :K]
    top_idx = jnp.take_along_axis(iidx, p2[:, :K], axis=1)

    _, top_idx, top_vals = lax.sort(
        (-top_vals, top_idx, top_vals), dimension=1, num_keys=2)
    top_idx = top_idx + jnp.asarray(topk - topk, dtype=top_idx.dtype)
    return top_vals, top_idx
